# baseline (speedup 1.0000x reference)
import functools

import jax
import jax.numpy as jnp
from jax import lax
from jax.experimental import pallas as pl
from jax.experimental.pallas import tpu as pltpu
from jax.experimental.pallas import tpu_sc as plsc

_NC = 2
_NS = 16
_NW = _NC * _NS
_LANES = 16

_CHUNK = 64
_DEPTH = 2


def _pack_pairs(x):
    lo = lax.bitcast_convert_type(
        x[:, :64].astype(jnp.bfloat16), jnp.uint16).astype(jnp.uint32)
    hi = lax.bitcast_convert_type(
        x[:, 64:].astype(jnp.bfloat16), jnp.uint16).astype(jnp.uint32)
    return lax.bitcast_convert_type((hi << 16) | lo, jnp.int32)


def _proj_body(wta_ref, wtb_ref, w_ref, out_ref):
    pa = lax.dot_general(wta_ref[...], w_ref[...], (((1,), (1,)), ((), ())),
                         preferred_element_type=jnp.float32)
    pb = lax.dot_general(wtb_ref[...], w_ref[...], (((1,), (1,)), ((), ())),
                         preferred_element_type=jnp.float32)
    out_ref[...] = jnp.concatenate([_pack_pairs(pa), _pack_pairs(pb)], axis=1)


def _project_table(word_table, W_word):
    V, D = word_table.shape
    E = W_word.shape[0]
    H = V // 2
    R = 4000
    assert H % R == 0
    nblk = H // R
    return pl.pallas_call(
        _proj_body,
        grid=(nblk,),
        in_specs=[
            pl.BlockSpec((R, D), lambda i: (i, 0)),
            pl.BlockSpec((R, D), lambda i: (i + nblk, 0)),
            pl.BlockSpec((E, D), lambda i: (0, 0)),
        ],
        out_specs=pl.BlockSpec((R, E), lambda i: (i, 0)),
        out_shape=jax.ShapeDtypeStruct((H, E), jnp.int32),
    )(word_table, word_table, W_word)


def _sc_mix_body(proj_hbm, ctab_hbm, midx_hbm, parr_hbm, cidx_hbm, out_hbm,
                 midx_v, parr_v, cidx_v, wrows_v, crows_v,
                 sem_idx, sem_g0, sem_g1, sem_s0, sem_s1):
    n_tok = out_hbm.shape[0]
    per_w = n_tok // _NW
    n_chunks = per_w // _CHUNK
    wid = lax.axis_index("s") * _NC + lax.axis_index("c")
    base = wid * per_w
    sem_g = (sem_g0, sem_g1)
    sem_s = (sem_s0, sem_s1)

    parr_dst = parr_v.at[pl.ds(0, per_w)]
    pltpu.async_copy(midx_hbm.at[wid], midx_v, sem_idx)
    pltpu.async_copy(parr_hbm.at[wid], parr_dst, sem_idx)
    pltpu.async_copy(cidx_hbm.at[wid], cidx_v, sem_idx).wait()
    pltpu.make_async_copy(midx_hbm.at[wid], midx_v, sem_idx).wait()
    pltpu.make_async_copy(parr_hbm.at[wid], parr_dst, sem_idx).wait()

    def issue_gathers(g, slot):
        im = midx_v.at[pl.ds(g * _CHUNK, _CHUNK)]
        ic = cidx_v.at[pl.ds(g * _CHUNK, _CHUNK)]
        pltpu.async_copy(proj_hbm.at[im], wrows_v.at[slot], sem_g[slot])
        pltpu.async_copy(ctab_hbm.at[ic], crows_v.at[slot], sem_g[slot])

    def wait_gathers(g, slot):
        im = midx_v.at[pl.ds(g * _CHUNK, _CHUNK)]
        ic = cidx_v.at[pl.ds(g * _CHUNK, _CHUNK)]
        pltpu.make_async_copy(proj_hbm.at[im], wrows_v.at[slot],
                              sem_g[slot]).wait()
        pltpu.make_async_copy(ctab_hbm.at[ic], crows_v.at[slot],
                              sem_g[slot]).wait()

    def out_copy(g, slot):
        off = pl.multiple_of(base + g * _CHUNK, _CHUNK)
        return pltpu.make_async_copy(crows_v.at[slot],
                                     out_hbm.at[pl.ds(off, _CHUNK)], sem_s[slot])

    issue_gathers(0, 0)

    def outer(o, carry):
        for b in range(_DEPTH):
            g = _DEPTH * o + b

            @pl.when(g + 1 < n_chunks)
            def _():
                @pl.when(g >= 1)
                def _():
                    out_copy(g - 1, 1 - b).wait()
                issue_gathers(g + 1, 1 - b)

            wait_gathers(g, b)

            def add_row(r, c2):
                hv = parr_v[pl.ds(g * _CHUNK + r, _LANES)]
                h = hv[0]
                for w in range(4):
                    sl = pl.ds(w * _LANES, _LANES)
                    vw = wrows_v[b, r, pl.ds(h + w * _LANES, _LANES)]
                    lo = lax.bitcast_convert_type(vw << 16, jnp.float32)
                    hi = lax.bitcast_convert_type(vw & jnp.int32(-65536),
                                                  jnp.float32)
                    sh = pl.ds(64 + w * _LANES, _LANES)
                    crows_v[b, r, sl] = crows_v[b, r, sl] + lo
                    crows_v[b, r, sh] = crows_v[b, r, sh] + hi
                return c2

            lax.fori_loop(0, _CHUNK, add_row, 0, unroll=False)
            out_copy(g, b).start()
        return carry

    lax.fori_loop(0, n_chunks // _DEPTH, outer, 0, unroll=False)
    out_copy(n_chunks - 2, 0).wait()
    out_copy(n_chunks - 1, 1).wait()


def _sc_mix(proj_pk, char_table, midx, parr, cidx):
    n_tok = midx.shape[0]
    E = char_table.shape[1]
    per_w = n_tok // _NW
    return pl.kernel(
        _sc_mix_body,
        out_type=jax.ShapeDtypeStruct((n_tok, E), jnp.float32),
        mesh=plsc.VectorSubcoreMesh(core_axis_name="c", subcore_axis_name="s",
                                    num_cores=_NC, num_subcores=_NS),
        scratch_types=[
            pltpu.VMEM((per_w,), jnp.int32),
            pltpu.VMEM((per_w + _LANES,), jnp.int32),
            pltpu.VMEM((per_w,), jnp.int32),
            pltpu.VMEM((_DEPTH, _CHUNK, E), jnp.int32),
            pltpu.VMEM((_DEPTH, _CHUNK, E), jnp.float32),
        ] + [pltpu.SemaphoreType.DMA] * 5,
    )(proj_pk, char_table, midx.reshape(_NW, per_w), parr.reshape(_NW, per_w),
      cidx.reshape(_NW, per_w))


def kernel(word, char, word_table, char_table, W_word):
    B, L = word.shape
    E = W_word.shape[0]
    proj_pk = _project_table(word_table, W_word)
    widx = word.reshape(-1).astype(jnp.int32)
    cidx = char.reshape(-1).astype(jnp.int32)
    half = word_table.shape[0] // 2
    midx = widx % half
    parr = (widx // half) * 64
    out = _sc_mix(proj_pk, char_table, midx, parr, cidx)
    return out.reshape(B, L, E)


# speedup vs baseline: 1.8833x; 1.8833x over previous
import functools

import jax
import jax.numpy as jnp
from jax import lax
from jax.experimental import pallas as pl
from jax.experimental.pallas import tpu as pltpu
from jax.experimental.pallas import tpu_sc as plsc

_NC = 2
_NS = 16
_NW = _NC * _NS
_LANES = 16

_CHUNK = 64
_DEPTH = 4
_AHEAD = 2


def _proj_body(wt_ref, w_ref, out_ref):
    out_ref[...] = lax.dot_general(
        wt_ref[...], w_ref[...], (((1,), (1,)), ((), ())),
        preferred_element_type=jnp.float32)


def _project_table(word_table, W_word):
    V, D = word_table.shape
    E = W_word.shape[0]
    R = 8000
    assert V % R == 0
    return pl.pallas_call(
        _proj_body,
        grid=(V // R,),
        in_specs=[
            pl.BlockSpec((R, D), lambda i: (i, 0)),
            pl.BlockSpec((E, D), lambda i: (0, 0)),
        ],
        out_specs=pl.BlockSpec((R, E), lambda i: (i, 0)),
        out_shape=jax.ShapeDtypeStruct((V, E), jnp.float32),
    )(word_table, W_word)


def _sc_mix_body(proj_hbm, ctab_hbm, widx_hbm, cidx_hbm, out_hbm,
                 idxw_v, idxc_v, wrows_v, crows_v,
                 sem_idx, sem_g0, sem_g1, sem_g2, sem_g3,
                 sem_s0, sem_s1, sem_s2, sem_s3):
    n_tok = out_hbm.shape[0]
    per_w = n_tok // _NW
    n_chunks = per_w // _CHUNK
    wid = lax.axis_index("s") * _NC + lax.axis_index("c")
    base = wid * per_w
    sem_g = (sem_g0, sem_g1, sem_g2, sem_g3)
    sem_s = (sem_s0, sem_s1, sem_s2, sem_s3)

    pltpu.async_copy(widx_hbm.at[wid], idxw_v, sem_idx)
    pltpu.async_copy(cidx_hbm.at[wid], idxc_v, sem_idx).wait()
    pltpu.make_async_copy(widx_hbm.at[wid], idxw_v, sem_idx).wait()

    def issue_gathers(g, slot):
        iw = idxw_v.at[pl.ds(g * _CHUNK, _CHUNK)]
        ic = idxc_v.at[pl.ds(g * _CHUNK, _CHUNK)]
        pltpu.async_copy(proj_hbm.at[iw], wrows_v.at[slot], sem_g[slot])
        pltpu.async_copy(ctab_hbm.at[ic], crows_v.at[slot], sem_g[slot])

    def wait_gathers(g, slot):
        iw = idxw_v.at[pl.ds(g * _CHUNK, _CHUNK)]
        ic = idxc_v.at[pl.ds(g * _CHUNK, _CHUNK)]
        pltpu.make_async_copy(proj_hbm.at[iw], wrows_v.at[slot],
                              sem_g[slot]).wait()
        pltpu.make_async_copy(ctab_hbm.at[ic], crows_v.at[slot],
                              sem_g[slot]).wait()

    def out_copy(g, slot):
        off = pl.multiple_of(base + g * _CHUNK, _CHUNK)
        return pltpu.make_async_copy(wrows_v.at[slot],
                                     out_hbm.at[pl.ds(off, _CHUNK)], sem_s[slot])

    for g0 in range(_AHEAD):
        issue_gathers(g0, g0 % _DEPTH)

    def outer(o, carry):
        for b in range(_DEPTH):
            g = _DEPTH * o + b
            sa = (b + _AHEAD) % _DEPTH
            @pl.when(g + _AHEAD < n_chunks)
            def _():
                @pl.when(g + _AHEAD >= _DEPTH)
                def _():
                    out_copy(g + _AHEAD - _DEPTH, sa).wait()
                issue_gathers(g + _AHEAD, sa)

            wait_gathers(g, b)

            def add_row(r, c2):
                for j in range(8):
                    sl = pl.ds(j * _LANES, _LANES)
                    wrows_v[b, r, sl] = wrows_v[b, r, sl] + crows_v[b, r, sl]
                return c2

            lax.fori_loop(0, _CHUNK, add_row, 0, unroll=False)
            out_copy(g, b).start()
        return carry

    lax.fori_loop(0, n_chunks // _DEPTH, outer, 0, unroll=False)
    for g0 in range(n_chunks - _DEPTH, n_chunks):
        out_copy(g0, g0 % _DEPTH).wait()


def _sc_mix(proj, char_table, widx, cidx):
    n_tok = widx.shape[0]
    E = proj.shape[1]
    per_w = n_tok // _NW
    return pl.kernel(
        _sc_mix_body,
        out_type=jax.ShapeDtypeStruct((n_tok, E), jnp.float32),
        mesh=plsc.VectorSubcoreMesh(core_axis_name="c", subcore_axis_name="s",
                                    num_cores=_NC, num_subcores=_NS),
        scratch_types=[
            pltpu.VMEM((per_w,), jnp.int32),
            pltpu.VMEM((per_w,), jnp.int32),
            pltpu.VMEM((_DEPTH, _CHUNK, E), jnp.float32),
            pltpu.VMEM((_DEPTH, _CHUNK, E), jnp.float32),
        ] + [pltpu.SemaphoreType.DMA] * 9,
    )(proj, char_table, widx.reshape(_NW, per_w), cidx.reshape(_NW, per_w))


def kernel(word, char, word_table, char_table, W_word):
    B, L = word.shape
    E = W_word.shape[0]
    proj = _project_table(word_table, W_word)
    widx = word.reshape(-1).astype(jnp.int32)
    cidx = char.reshape(-1).astype(jnp.int32)
    out = _sc_mix(proj, char_table, widx, cidx)
    return out.reshape(B, L, E)


# speedup vs baseline: 1.8941x; 1.0058x over previous
import functools

import jax
import jax.numpy as jnp
from jax import lax
from jax.experimental import pallas as pl
from jax.experimental.pallas import tpu as pltpu
from jax.experimental.pallas import tpu_sc as plsc

_NC = 2
_NS = 16
_NW = _NC * _NS
_LANES = 16

_CHUNK = 64
_DEPTH = 4
_AHEAD = 2


def _proj_body(wt_ref, w_ref, out_ref):
    out_ref[...] = lax.dot_general(
        wt_ref[...], w_ref[...], (((1,), (1,)), ((), ())),
        preferred_element_type=jnp.float32)


def _project_table(word_table, W_word):
    V, D = word_table.shape
    E = W_word.shape[0]
    R = 20000
    assert V % R == 0
    return pl.pallas_call(
        _proj_body,
        grid=(V // R,),
        in_specs=[
            pl.BlockSpec((R, D), lambda i: (i, 0)),
            pl.BlockSpec((E, D), lambda i: (0, 0)),
        ],
        out_specs=pl.BlockSpec((R, E), lambda i: (i, 0)),
        out_shape=jax.ShapeDtypeStruct((V, E), jnp.float32),
    )(word_table, W_word)


def _sc_mix_body(proj_hbm, ctab_hbm, widx_hbm, cidx_hbm, out_hbm,
                 idxw_v, idxc_v, wrows_v, crows_v,
                 sem_idx, sem_g0, sem_g1, sem_g2, sem_g3,
                 sem_s0, sem_s1, sem_s2, sem_s3):
    n_tok = out_hbm.shape[0]
    per_w = n_tok // _NW
    n_chunks = per_w // _CHUNK
    wid = lax.axis_index("s") * _NC + lax.axis_index("c")
    base = wid * per_w
    sem_g = (sem_g0, sem_g1, sem_g2, sem_g3)
    sem_s = (sem_s0, sem_s1, sem_s2, sem_s3)

    pltpu.async_copy(widx_hbm.at[wid], idxw_v, sem_idx)
    pltpu.async_copy(cidx_hbm.at[wid], idxc_v, sem_idx).wait()
    pltpu.make_async_copy(widx_hbm.at[wid], idxw_v, sem_idx).wait()

    def issue_gathers(g, slot):
        iw = idxw_v.at[pl.ds(g * _CHUNK, _CHUNK)]
        ic = idxc_v.at[pl.ds(g * _CHUNK, _CHUNK)]
        pltpu.async_copy(proj_hbm.at[iw], wrows_v.at[slot], sem_g[slot])
        pltpu.async_copy(ctab_hbm.at[ic], crows_v.at[slot], sem_g[slot])

    def wait_gathers(g, slot):
        iw = idxw_v.at[pl.ds(g * _CHUNK, _CHUNK)]
        ic = idxc_v.at[pl.ds(g * _CHUNK, _CHUNK)]
        pltpu.make_async_copy(proj_hbm.at[iw], wrows_v.at[slot],
                              sem_g[slot]).wait()
        pltpu.make_async_copy(ctab_hbm.at[ic], crows_v.at[slot],
                              sem_g[slot]).wait()

    def out_copy(g, slot):
        off = pl.multiple_of(base + g * _CHUNK, _CHUNK)
        return pltpu.make_async_copy(wrows_v.at[slot],
                                     out_hbm.at[pl.ds(off, _CHUNK)], sem_s[slot])

    for g0 in range(_AHEAD):
        issue_gathers(g0, g0 % _DEPTH)

    def outer(o, carry):
        for b in range(_DEPTH):
            g = _DEPTH * o + b
            sa = (b + _AHEAD) % _DEPTH
            @pl.when(g + _AHEAD < n_chunks)
            def _():
                @pl.when(g + _AHEAD >= _DEPTH)
                def _():
                    out_copy(g + _AHEAD - _DEPTH, sa).wait()
                issue_gathers(g + _AHEAD, sa)

            wait_gathers(g, b)

            def add_row(r, c2):
                for j in range(8):
                    sl = pl.ds(j * _LANES, _LANES)
                    wrows_v[b, r, sl] = wrows_v[b, r, sl] + crows_v[b, r, sl]
                return c2

            lax.fori_loop(0, _CHUNK, add_row, 0, unroll=False)
            out_copy(g, b).start()
        return carry

    lax.fori_loop(0, n_chunks // _DEPTH, outer, 0, unroll=False)
    for g0 in range(n_chunks - _DEPTH, n_chunks):
        out_copy(g0, g0 % _DEPTH).wait()


def _sc_mix(proj, char_table, widx, cidx):
    n_tok = widx.shape[0]
    E = proj.shape[1]
    per_w = n_tok // _NW
    return pl.kernel(
        _sc_mix_body,
        out_type=jax.ShapeDtypeStruct((n_tok, E), jnp.float32),
        mesh=plsc.VectorSubcoreMesh(core_axis_name="c", subcore_axis_name="s",
                                    num_cores=_NC, num_subcores=_NS),
        scratch_types=[
            pltpu.VMEM((per_w,), jnp.int32),
            pltpu.VMEM((per_w,), jnp.int32),
            pltpu.VMEM((_DEPTH, _CHUNK, E), jnp.float32),
            pltpu.VMEM((_DEPTH, _CHUNK, E), jnp.float32),
        ] + [pltpu.SemaphoreType.DMA] * 9,
    )(proj, char_table, widx.reshape(_NW, per_w), cidx.reshape(_NW, per_w))


def kernel(word, char, word_table, char_table, W_word):
    B, L = word.shape
    E = W_word.shape[0]
    proj = _project_table(word_table, W_word)
    widx = word.reshape(-1).astype(jnp.int32)
    cidx = char.reshape(-1).astype(jnp.int32)
    out = _sc_mix(proj, char_table, widx, cidx)
    return out.reshape(B, L, E)
